# baseline (speedup 1.0000x reference)
import functools

import jax
import jax.numpy as jnp
from jax import lax
from jax.experimental import pallas as pl
from jax.experimental.pallas import tpu as pltpu
from jax.experimental.pallas import tpu_sc as plsc

D_SIZE = 8
MC_SIZE = 8192
PQ_DIM = 64
BATCH = 16384

NC = 2
NS = 16
LANES = 16
NSEG = MC_SIZE // 128
NBB = BATCH // 128
JB = PQ_DIM // 8
J_PER_SC = JB // NC
NCHUNK = 16
BB_PER_CHUNK = NBB // NCHUNK
ZSLICE = NSEG * 8 * 128


def _pq_body(z_hbm, codes_hbm, out_hbm, zbuf, cbuf, psum, acc_sh, sem, sem_add):
    c = lax.axis_index("c")
    sid = lax.axis_index("s")
    jl = sid // 4
    jg = c * J_PER_SC + jl
    p = sid % 4

    zero = jnp.zeros((LANES,), jnp.float32)

    @plsc.parallel_loop(0, 4 * BB_PER_CHUNK, unroll=2)
    def _zero_row(r):
        for cg in range(512 // LANES):
            psum[r, pl.ds(cg * LANES, LANES)] = zero

    zrow = ((sid // 4) * NBB + (sid % 4) * 32) * 2
    pltpu.sync_copy(psum, acc_sh.at[pl.ds(zrow, 32)])
    pltpu.sync_copy(psum, acc_sh.at[pl.ds(zrow + 32, 32)])
    plsc.subcore_barrier()

    drain_src = out_hbm.at[0, pl.ds(0, 16)]

    def _task(t, carry_t):
        i = p * 2 + t
        pltpu.async_copy(z_hbm.at[i, jg], zbuf, sem).wait()
        pltpu.async_copy(codes_hbm.at[:, i], cbuf, sem).wait()

        def _chunk(qq, carry_q):
            for h in range(2):
                q = qq * 2 + h
                n = t * NCHUNK + q
                pb = h * 16

                @pl.when(n >= 2)
                def _():
                    pltpu.make_async_copy(
                        drain_src, psum.at[pl.ds(pb, 16)], sem_add
                    ).wait()

                @plsc.parallel_loop(0, BB_PER_CHUNK, unroll=2)
                def _bb_body(bb):
                    row = q * BB_PER_CHUNK + bb
                    for gl in range(128 // LANES):
                        code = cbuf[row, pl.ds(gl * LANES, LANES)]
                        addr = lax.bitwise_or(
                            lax.shift_left(
                                lax.shift_right_logical(code, 7), 10
                            ),
                            lax.bitwise_and(code, 127),
                        )
                        for s in range(8):
                            col = s * 128 + gl * LANES
                            v = plsc.load_gather(zbuf, [addr + (s * 128)])
                            psum[
                                pb + 2 * bb + col // 512,
                                pl.ds(col % 512, LANES),
                            ] = v * 0.125

                rows = jnp.arange(LANES, dtype=jnp.int32) + (
                    (jl * NBB + q * BB_PER_CHUNK) * 2
                )
                pltpu.async_copy(
                    psum.at[pl.ds(pb, 16)], acc_sh.at[rows], sem_add, add=True
                )
            return carry_q

        lax.fori_loop(0, NCHUNK // 2, _chunk, 0)
        return carry_t

    lax.fori_loop(0, 2, _task, 0)

    for _ in range(2):
        pltpu.make_async_copy(drain_src, psum.at[pl.ds(0, 16)], sem_add).wait()

    plsc.subcore_barrier()

    jzg = c * J_PER_SC + (sid // 4)
    pltpu.sync_copy(
        acc_sh.at[pl.ds(zrow, 64)],
        out_hbm.at[jzg, pl.ds((sid % 4) * 64, 64)],
    )


_pq_call = pl.kernel(
    _pq_body,
    out_type=jax.ShapeDtypeStruct((JB, NBB * 2, 512), jnp.float32),
    mesh=plsc.VectorSubcoreMesh(core_axis_name="c", subcore_axis_name="s"),
    scratch_types=[
        pltpu.VMEM((ZSLICE,), jnp.float32),
        pltpu.VMEM((NBB, 128), jnp.int32),
        pltpu.VMEM((4 * BB_PER_CHUNK, 512), jnp.float32),
        pltpu.VMEM_SHARED((J_PER_SC * NBB * 2, 512), jnp.float32),
        pltpu.SemaphoreType.DMA,
        pltpu.SemaphoreType.DMA,
    ],
    compiler_params=pltpu.CompilerParams(
        use_tc_tiling_on_sc=False, needs_layout_passes=False
    ),
)


@jax.jit
def kernel(code_list, tables):
    z = (
        tables.transpose(0, 2, 1)
        .reshape(D_SIZE, JB, 8, NSEG, 128)
        .transpose(0, 1, 3, 2, 4)
        .reshape(D_SIZE, JB, ZSLICE)
    )
    codes = code_list.astype(jnp.int32).reshape(D_SIZE, NBB, 128).transpose(1, 0, 2)
    out3 = _pq_call(z, codes)
    return (
        out3.reshape(JB, NBB, 8, 128)
        .transpose(1, 3, 0, 2)
        .reshape(BATCH, PQ_DIM)
    )


# speedup vs baseline: 1.2039x; 1.2039x over previous
import functools

import jax
import jax.numpy as jnp
from jax import lax
from jax.experimental import pallas as pl
from jax.experimental.pallas import tpu as pltpu
from jax.experimental.pallas import tpu_sc as plsc

D_SIZE = 8
MC_SIZE = 8192
PQ_DIM = 64
BATCH = 16384

NC = 2
NS = 16
LANES = 16
NSEG = MC_SIZE // 128
NBB = BATCH // 128
JB = PQ_DIM // 8
J_PER_SC = JB // NC
NCHUNK = 8
BB_PER_CHUNK = NBB // NCHUNK
ZSLICE = NSEG * 8 * 128


def _pq_body(z_hbm, codes_hbm, out_hbm, zbuf, cbuf, psum, acc_sh, sem, sem_add):
    c = lax.axis_index("c")
    sid = lax.axis_index("s")
    jl = sid // 4
    jg = c * J_PER_SC + jl
    p = sid % 4

    zero = jnp.zeros((LANES,), jnp.float32)

    @plsc.parallel_loop(0, BB_PER_CHUNK, unroll=2)
    def _zero_row(r):
        for cg in range(1024 // LANES):
            psum[r, pl.ds(cg * LANES, LANES)] = zero

    zrow = (sid // 4) * NBB + (sid % 4) * 32
    pltpu.sync_copy(psum, acc_sh.at[pl.ds(zrow, BB_PER_CHUNK)])
    pltpu.sync_copy(psum, acc_sh.at[pl.ds(zrow + 16, BB_PER_CHUNK)])
    plsc.subcore_barrier()

    def _task(t, carry_t):
        i = p * 2 + t
        pltpu.async_copy(z_hbm.at[i, jg], zbuf, sem).wait()
        pltpu.async_copy(codes_hbm.at[:, i], cbuf, sem).wait()

        def _chunk(q, carry_q):
            @plsc.parallel_loop(0, BB_PER_CHUNK, unroll=2)
            def _bb_body(bb):
                row = q * BB_PER_CHUNK + bb
                for gl in range(128 // LANES):
                    code = cbuf[row, pl.ds(gl * LANES, LANES)]
                    addr = lax.bitwise_or(
                        lax.shift_left(lax.shift_right_logical(code, 7), 10),
                        lax.bitwise_and(code, 127),
                    )
                    for s in range(8):
                        v = plsc.load_gather(zbuf, [addr + (s * 128)])
                        psum[bb, pl.ds(s * 128 + gl * LANES, LANES)] = v * 0.125

            rows = jnp.arange(BB_PER_CHUNK, dtype=jnp.int32) + (
                jl * NBB + q * BB_PER_CHUNK
            )
            pltpu.async_copy(psum, acc_sh.at[rows], sem_add, add=True).wait()
            return carry_q

        lax.fori_loop(0, NCHUNK, _chunk, 0)
        return carry_t

    lax.fori_loop(0, 2, _task, 0)

    plsc.subcore_barrier()

    jzg = c * J_PER_SC + (sid // 4)
    pltpu.sync_copy(
        acc_sh.at[pl.ds(zrow, 32)],
        out_hbm.at[jzg, pl.ds((sid % 4) * 32, 32)],
    )


_pq_call = pl.kernel(
    _pq_body,
    out_type=jax.ShapeDtypeStruct((JB, NBB, 1024), jnp.float32),
    mesh=plsc.VectorSubcoreMesh(core_axis_name="c", subcore_axis_name="s"),
    scratch_types=[
        pltpu.VMEM((ZSLICE,), jnp.float32),
        pltpu.VMEM((NBB, 128), jnp.int32),
        pltpu.VMEM((BB_PER_CHUNK, 1024), jnp.float32),
        pltpu.VMEM_SHARED((J_PER_SC * NBB, 1024), jnp.float32),
        pltpu.SemaphoreType.DMA,
        pltpu.SemaphoreType.DMA,
    ],
    compiler_params=pltpu.CompilerParams(
        use_tc_tiling_on_sc=False, needs_layout_passes=False
    ),
)


@jax.jit
def kernel(code_list, tables):
    z = (
        tables.transpose(0, 2, 1)
        .reshape(D_SIZE, JB, 8, NSEG, 128)
        .transpose(0, 1, 3, 2, 4)
        .reshape(D_SIZE, JB, ZSLICE)
    )
    codes = code_list.astype(jnp.int32).reshape(D_SIZE, NBB, 128).transpose(1, 0, 2)
    out3 = _pq_call(z, codes)
    return (
        out3.reshape(JB, NBB, 8, 128)
        .transpose(1, 3, 0, 2)
        .reshape(BATCH, PQ_DIM)
    )
